# Initial kernel scaffold; baseline (speedup 1.0000x reference)
import jax
import jax.numpy as jnp
from jax.experimental import pallas as pl


def kernel(x, table):
    raise NotImplementedError("write your pallas kernel here")


# speedup vs baseline: 3.5587x; 3.5587x over previous
import functools

import jax
import jax.numpy as jnp
import numpy as np
from jax import lax
from jax.experimental import pallas as pl
from jax.experimental.pallas import tpu as pltpu
from jax.experimental.pallas import tpu_sc as plsc

D_MODEL = 1024
BATCH = 4
SEQ = 4096
NUM_CORES = 2
NUM_SUBCORES = 16
NW = NUM_CORES * NUM_SUBCORES
ROWS = BATCH * SEQ
ROWS_PER_W = ROWS // NW
CHUNK = 16
NUM_CHUNKS = ROWS_PER_W // CHUNK
LANES = 16
SLICES = D_MODEL // LANES
SCALE = 32.0


def _position_encoding(seq_len: int, d_model: int) -> np.ndarray:
    pos = np.arange(seq_len, dtype=np.float64)[:, None]
    two_i = np.arange(0, d_model, 2, dtype=np.float64)
    div = np.power(10000.0, two_i / d_model)
    enc = np.zeros((seq_len, d_model), dtype=np.float64)
    enc[:, 0::2] = np.sin(pos / div)
    enc[:, 1::2] = np.cos(pos / div)
    return enc.astype(np.float32)


_PE = _position_encoding(SEQ, D_MODEL)

_MESH = plsc.VectorSubcoreMesh(
    core_axis_name="c", subcore_axis_name="s",
    num_cores=NUM_CORES, num_subcores=NUM_SUBCORES,
)


@functools.partial(
    pl.kernel,
    out_type=jax.ShapeDtypeStruct((ROWS, D_MODEL), jnp.float32),
    mesh=_MESH,
    scratch_types=[
        pltpu.VMEM((CHUNK,), jnp.int32),
        pltpu.VMEM((CHUNK, D_MODEL), jnp.float32),
        pltpu.VMEM((CHUNK, D_MODEL), jnp.float32),
        pltpu.SemaphoreType.DMA,
        pltpu.SemaphoreType.DMA,
    ],
)
def _embed(x_hbm, table_hbm, pe_hbm, out_hbm, idx_v, rows_v, pos_v, gsem, psem):
    wid = lax.axis_index("s") * NUM_CORES + lax.axis_index("c")
    base = wid * ROWS_PER_W
    pbase = lax.rem(base, SEQ)
    scale = jnp.float32(SCALE)

    def chunk_body(c, carry):
        r0 = base + c * CHUNK
        p0 = pbase + c * CHUNK
        pltpu.sync_copy(x_hbm.at[pl.ds(r0, CHUNK)], idx_v)
        g = pltpu.async_copy(table_hbm.at[idx_v], rows_v, gsem)
        p = pltpu.async_copy(pe_hbm.at[pl.ds(p0, CHUNK)], pos_v, psem)
        g.wait()
        p.wait()

        def row_body(j, rcarry):
            for i in range(SLICES):
                sl = pl.ds(i * LANES, LANES)
                rows_v[j, sl] = rows_v[j, sl] * scale + pos_v[j, sl]
            return rcarry

        lax.fori_loop(0, CHUNK, row_body, 0)
        pltpu.sync_copy(rows_v, out_hbm.at[pl.ds(r0, CHUNK)])
        return carry

    lax.fori_loop(0, NUM_CHUNKS, chunk_body, 0)


def kernel(x, table):
    pe = jnp.asarray(_PE)
    out = _embed(x.reshape(ROWS).astype(jnp.int32), table, pe)
    return out.reshape(BATCH, SEQ, D_MODEL)
